# baseline (speedup 1.0000x reference)
import numpy as np
import jax
import jax.numpy as jnp
from jax.experimental import pallas as pl
from jax.experimental.pallas import tpu as pltpu

_NB = 512
_NB1 = 1024


def _dot(a, b):
    return jnp.dot(a.astype(jnp.bfloat16), b.astype(jnp.bfloat16),
                   preferred_element_type=jnp.float32)


def _sigmoid(x):
    return 0.5 * jnp.tanh(0.5 * x) + 0.5


def _pool_conv_selector(H):
    W = H
    H2, W2 = H // 2, W // 2
    S = np.zeros((H * W, 4 * H2 * W2, 9), np.float32)
    for ph in range(2):
        for pw in range(2):
            q = 2 * ph + pw
            for oh in range(H2):
                for ow in range(W2):
                    col = q * H2 * W2 + oh * W2 + ow
                    for dh in range(3):
                        for dw in range(3):
                            ih = 2 * oh + ph + dh - 1
                            iw = 2 * ow + pw + dw - 1
                            if 0 <= ih < H and 0 <= iw < W:
                                S[ih * W + iw, col, dh * 3 + dw] = 1.0
    return S


def _conv_big_weight(w, H):
    Cin, Cout = w.shape[2], w.shape[3]
    S = _pool_conv_selector(H)
    P, Q = S.shape[0], S.shape[1]
    wb = w.astype(jnp.bfloat16)
    if Cin == 1:
        S2 = jnp.asarray(S.reshape(P * Q, 9), jnp.bfloat16)
        big = jnp.dot(S2, wb.reshape(9, Cout),
                      preferred_element_type=jnp.bfloat16)
        return big.reshape(P, Q * Cout)
    K = np.einsum('pqt,cd->pcqtd', S, np.eye(Cin, dtype=np.float32))
    K2 = jnp.asarray(K.reshape(P * Cin * Q, 9 * Cin), jnp.bfloat16)
    big = jnp.dot(K2, wb.reshape(9 * Cin, Cout),
                  preferred_element_type=jnp.bfloat16)
    return big.reshape(P * Cin, Q * Cout)


def _quad_max(y, M):
    return jnp.maximum(jnp.maximum(y[:, :M], y[:, M:2 * M]),
                       jnp.maximum(y[:, 2 * M:3 * M], y[:, 3 * M:]))


def _encode_body(x_ref, pos_ref,
                 cw1, cb1, cw2, cb2, cw3, cb3,
                 lf0, lb0, lf1, lb1, lf2, lb2, lf3, lb3,
                 lr0, rb0, lr1, rb1, lr2, rb2, lr3, rb3,
                 m0w, m0b, m1w, m1b, m2w, m2b, m3w, m3b,
                 g1w_ref, g2w_ref,
                 p0_ref, xw1_ref, xw2c_ref):
    h = x_ref[...]
    for cw, cb, M in ((cw1, cb1, 512), (cw2, cb2, 256), (cw3, cb3, 128)):
        y = _dot(h, cw[...])
        cbt = jnp.tile(cb[...], (1, M // cb.shape[1]))
        h = (_quad_max(y, M) + cbt).astype(jnp.bfloat16)

    def lstm(stack):
        hh = h.astype(jnp.float32)
        for w_ref, b_ref in stack:
            cin = hh.shape[1]
            z = _dot(hh, w_ref[0, 0][:cin, :]) + b_ref[...]
            i = _sigmoid(z[:, 0:64])
            og = _sigmoid(z[:, 128:192])
            g = jnp.tanh(z[:, 192:256])
            hh = og * jnp.tanh(i * g)
        return hh

    hf = lstm(((lf0, lb0), (lf1, lb1), (lf2, lb2), (lf3, lb3)))
    hr = lstm(((lr0, rb0), (lr1, rb1), (lr2, rb2), (lr3, rb3)))

    p = pos_ref[...]
    for w_ref, b_ref in ((m0w, m0b), (m1w, m1b), (m2w, m2b), (m3w, m3b)):
        p = jnp.maximum(_dot(p, w_ref[...]) + b_ref[...], 0.0)

    g1w = g1w_ref[...]
    xw1 = _dot(p, g1w[0:32]) + _dot(hf, g1w[32:96]) + _dot(hr, g1w[96:160])
    g2w = g2w_ref[...]
    xw2c = _dot(hf, g2w[32:96]) + _dot(hr, g2w[96:160])

    p0_ref[...] = p
    xw1_ref[...] = xw1.astype(jnp.bfloat16)
    xw2c_ref[...] = xw2c.astype(jnp.bfloat16)


def _gcn1_body(a_ref, xw_ref, p0_ref, xw2c_ref,
               b1_ref, w1w_ref, b1w_ref, g2wp_ref,
               p1_ref, xw2_ref):
    t = _dot(a_ref[...], xw_ref[...]) + b1_ref[...]
    feats = jnp.maximum(_dot(t, w1w_ref[...]) + b1w_ref[...], 0.0)
    p1 = feats + p0_ref[...]
    xw2 = _dot(p1, g2wp_ref[...][0:32]) + xw2c_ref[...].astype(jnp.float32)
    p1_ref[...] = p1
    xw2_ref[...] = xw2.astype(jnp.bfloat16)


def _gcn2_body(a_ref, xw_ref, p1_ref,
               b2_ref, w2w_ref, b2w_ref, cls_w_ref, cls_b_ref,
               o_ref):
    t = _dot(a_ref[...], xw_ref[...]) + b2_ref[...]
    feats = jnp.maximum(_dot(t, w2w_ref[...]) + b2w_ref[...], 0.0)
    p2 = feats + p1_ref[...]
    o_ref[...] = _dot(p2, cls_w_ref[...]) + cls_b_ref[...]


def kernel(pos_features, im_patches, A,
           pos_mlp_0_w, pos_mlp_0_b, pos_mlp_1_w, pos_mlp_1_b,
           pos_mlp_2_w, pos_mlp_2_b, pos_mlp_3_w, pos_mlp_3_b,
           pix_0_w, pix_0_b, pix_1_w, pix_1_b, pix_2_w, pix_2_b,
           bf_0_w, bf_0_b, bf_1_w, bf_1_b, bf_2_w, bf_2_b, bf_3_w, bf_3_b,
           br_0_w, br_0_b, br_1_w, br_1_b, br_2_w, br_2_b, br_3_w, br_3_b,
           gcn1_mw, gcn1_mb, gcn1w_mw, gcn1w_mb,
           gcn2_mw, gcn2_mb, gcn2w_mw, gcn2w_mb,
           cls_w, cls_b):
    N = pos_features.shape[0]
    sub = N // 2 // _NB1
    grid2 = (2, sub)
    par2 = pltpu.CompilerParams(
        dimension_semantics=("parallel", "arbitrary"))
    grid = (N // _NB,)
    par = pltpu.CompilerParams(dimension_semantics=("parallel",))

    def _rows2(shape):
        return pl.BlockSpec(
            shape, lambda i, j: (i * sub + j,) + (0,) * (len(shape) - 1))

    def _whole2(shape):
        return pl.BlockSpec(shape, lambda i, j: (0,) * len(shape))

    def _tap2(shape):
        return pl.BlockSpec((1, 1) + shape[2:], lambda i, j: (1, 1, 0, 0))

    def _rows(shape):
        return pl.BlockSpec(shape, lambda i: (i,) + (0,) * (len(shape) - 1))

    def _whole(shape):
        return pl.BlockSpec(shape, lambda i: (0,) * len(shape))

    x_im = im_patches.reshape(N, 64)
    cws = [_conv_big_weight(pix_0_w, 8),
           _conv_big_weight(pix_1_w, 4),
           _conv_big_weight(pix_2_w, 2)]
    cbs = [pix_0_b.reshape(1, 32), pix_1_b.reshape(1, 64),
           pix_2_b.reshape(1, 128)]
    lstm_w = [bf_0_w, bf_1_w, bf_2_w, bf_3_w,
              br_0_w, br_1_w, br_2_w, br_3_w]
    lstm_b = [b.reshape(1, 256) for b in (bf_0_b, bf_1_b, bf_2_b, bf_3_b,
                                          br_0_b, br_1_b, br_2_b, br_3_b)]
    mlp = [pos_mlp_0_w, pos_mlp_0_b.reshape(1, -1),
           pos_mlp_1_w, pos_mlp_1_b.reshape(1, -1),
           pos_mlp_2_w, pos_mlp_2_b.reshape(1, -1),
           pos_mlp_3_w, pos_mlp_3_b.reshape(1, -1)]

    args1 = [x_im, pos_features,
             cws[0], cbs[0], cws[1], cbs[1], cws[2], cbs[2]]
    for w, b in zip(lstm_w, lstm_b):
        args1 += [w, b]
    args1 += mlp + [gcn1_mw, gcn2_mw]
    specs1 = [_rows2((_NB1, 64)), _rows2((_NB1, 128))]
    specs1 += [_tap2(a.shape) if a.ndim == 4 else _whole2(a.shape)
               for a in args1[2:]]
    p0, xw1, xw2c = pl.pallas_call(
        _encode_body,
        grid=grid2,
        in_specs=specs1,
        out_specs=[_rows2((_NB1, 32)), _rows2((_NB1, 256)), _rows2((_NB1, 256))],
        out_shape=[jax.ShapeDtypeStruct((N, 32), jnp.float32),
                   jax.ShapeDtypeStruct((N, 256), jnp.bfloat16),
                   jax.ShapeDtypeStruct((N, 256), jnp.bfloat16)],
        compiler_params=par2,
    )(*args1)

    p1, xw2 = pl.pallas_call(
        _gcn1_body,
        grid=grid,
        in_specs=[_rows((_NB, N)), _whole((N, 256)), _rows((_NB, 32)),
                  _rows((_NB, 256)), _whole((1, 256)), _whole((256, 32)),
                  _whole((1, 32)), _whole((160, 256))],
        out_specs=[_rows((_NB, 32)), _rows((_NB, 256))],
        out_shape=[jax.ShapeDtypeStruct((N, 32), jnp.float32),
                   jax.ShapeDtypeStruct((N, 256), jnp.bfloat16)],
        compiler_params=par,
    )(A, xw1, p0, xw2c, gcn1_mb.reshape(1, 256), gcn1w_mw,
      gcn1w_mb.reshape(1, 32), gcn2_mw)

    out = pl.pallas_call(
        _gcn2_body,
        grid=grid,
        in_specs=[_rows((_NB, N)), _whole((N, 256)), _rows((_NB, 32)),
                  _whole((1, 256)), _whole((256, 32)), _whole((1, 32)),
                  _whole((32, 128)), _whole((1, 128))],
        out_specs=_rows((_NB, 128)),
        out_shape=jax.ShapeDtypeStruct((N, 128), jnp.float32),
        compiler_params=par,
    )(A, xw2, p1, gcn2_mb.reshape(1, 256), gcn2w_mw,
      gcn2w_mb.reshape(1, 32), cls_w, cls_b.reshape(1, 128))
    return out


# speedup vs baseline: 1.0668x; 1.0668x over previous
import functools
import numpy as np
import jax
import jax.numpy as jnp
from jax.experimental import pallas as pl
from jax.experimental.pallas import tpu as pltpu

_NB = 512
_NB1 = 1024


def _dot(a, b):
    return jnp.dot(a.astype(jnp.bfloat16), b.astype(jnp.bfloat16),
                   preferred_element_type=jnp.float32)


def _sigmoid(x):
    return 0.5 * jnp.tanh(0.5 * x) + 0.5


def _pool_conv_selector(H):
    W = H
    H2, W2 = H // 2, W // 2
    S = np.zeros((H * W, 4 * H2 * W2, 9), np.float32)
    for ph in range(2):
        for pw in range(2):
            q = 2 * ph + pw
            for oh in range(H2):
                for ow in range(W2):
                    col = q * H2 * W2 + oh * W2 + ow
                    for dh in range(3):
                        for dw in range(3):
                            ih = 2 * oh + ph + dh - 1
                            iw = 2 * ow + pw + dw - 1
                            if 0 <= ih < H and 0 <= iw < W:
                                S[ih * W + iw, col, dh * 3 + dw] = 1.0
    return S


def _conv_big_weight(w, H):
    Cin, Cout = w.shape[2], w.shape[3]
    S = _pool_conv_selector(H)
    Q = S.shape[1]
    wb = w.astype(jnp.bfloat16)
    if Cin == 1:
        S2 = jnp.asarray(S.reshape(H * H * Q, 9), jnp.bfloat16)
        big = jnp.dot(S2, wb.reshape(9, Cout),
                      preferred_element_type=jnp.bfloat16)
        return big.reshape(H * H, Q * Cout)
    Sb = jnp.asarray(S.reshape(H * H, Q, 3, 3), jnp.bfloat16)
    big = jnp.einsum('pqyx,yxcf->pcqf', Sb, wb,
                     preferred_element_type=jnp.bfloat16)
    return big.reshape(H * H * Cin, Q * Cout)


def _quad_max(y, M):
    return jnp.maximum(jnp.maximum(y[:, :M], y[:, M:2 * M]),
                       jnp.maximum(y[:, 2 * M:3 * M], y[:, 3 * M:]))


def _encode_body(x_ref, pos_ref,
                 cw1, cb1, cw2, cb2, cw3, cb3,
                 lf0, lb0, lf1, lb1, lf2, lb2, lf3, lb3,
                 lr0, rb0, lr1, rb1, lr2, rb2, lr3, rb3,
                 m0w, m0b, m1w, m1b, m2w, m2b, m3w, m3b,
                 cond_ref, p0_ref):
    h = x_ref[...]
    for cw, cb, M in ((cw1, cb1, 512), (cw2, cb2, 256), (cw3, cb3, 128)):
        y = _dot(h, cw[...])
        cbt = jnp.tile(cb[...], (1, M // cb.shape[1]))
        h = (_quad_max(y, M) + cbt).astype(jnp.bfloat16)

    def lstm(stack):
        hh = h.astype(jnp.float32)
        for w_ref, b_ref in stack:
            cin = hh.shape[1]
            z = _dot(hh, w_ref[0, 0][:cin, :]) + b_ref[...]
            i = _sigmoid(z[:, 0:64])
            og = _sigmoid(z[:, 128:192])
            g = jnp.tanh(z[:, 192:256])
            hh = og * jnp.tanh(i * g)
        return hh

    hf = lstm(((lf0, lb0), (lf1, lb1), (lf2, lb2), (lf3, lb3)))
    hr = lstm(((lr0, rb0), (lr1, rb1), (lr2, rb2), (lr3, rb3)))

    p = pos_ref[...]
    for w_ref, b_ref in ((m0w, m0b), (m1w, m1b), (m2w, m2b), (m3w, m3b)):
        p = jnp.maximum(_dot(p, w_ref[...]) + b_ref[...], 0.0)

    cond_ref[:, 0:64] = hf.astype(jnp.bfloat16)
    cond_ref[:, 64:128] = hr.astype(jnp.bfloat16)
    p0_ref[...] = p


def _gcn1_body(a_ref, p0_ref, cond_ref, g1w_ref,
               b1_ref, w1w_ref, b1w_ref,
               p1_ref, uc_ref, *, nb):
    i = pl.program_id(0)
    a = a_ref[...]
    u0 = _dot(a, p0_ref[...])
    uc = _dot(a, cond_ref[...])
    g1w = g1w_ref[...]
    t = _dot(u0, g1w[0:32]) + _dot(uc, g1w[32:160]) + b1_ref[...]
    feats = jnp.maximum(_dot(t, w1w_ref[...]) + b1w_ref[...], 0.0)
    p1 = feats + p0_ref[pl.ds(i * nb, nb), :]
    p1_ref[...] = p1
    uc_ref[...] = uc.astype(jnp.bfloat16)


def _gcn2_body(a_ref, p1_ref, uc_ref, g2w_ref,
               b2_ref, w2w_ref, b2w_ref, cls_w_ref, cls_b_ref,
               o_ref, *, nb):
    i = pl.program_id(0)
    u1 = _dot(a_ref[...], p1_ref[...])
    g2w = g2w_ref[...]
    t = (_dot(u1, g2w[0:32]) + _dot(uc_ref[...], g2w[32:160]) + b2_ref[...])
    feats = jnp.maximum(_dot(t, w2w_ref[...]) + b2w_ref[...], 0.0)
    p2 = feats + p1_ref[pl.ds(i * nb, nb), :]
    o_ref[...] = _dot(p2, cls_w_ref[...]) + cls_b_ref[...]


def kernel(pos_features, im_patches, A,
           pos_mlp_0_w, pos_mlp_0_b, pos_mlp_1_w, pos_mlp_1_b,
           pos_mlp_2_w, pos_mlp_2_b, pos_mlp_3_w, pos_mlp_3_b,
           pix_0_w, pix_0_b, pix_1_w, pix_1_b, pix_2_w, pix_2_b,
           bf_0_w, bf_0_b, bf_1_w, bf_1_b, bf_2_w, bf_2_b, bf_3_w, bf_3_b,
           br_0_w, br_0_b, br_1_w, br_1_b, br_2_w, br_2_b, br_3_w, br_3_b,
           gcn1_mw, gcn1_mb, gcn1w_mw, gcn1w_mb,
           gcn2_mw, gcn2_mb, gcn2w_mw, gcn2w_mb,
           cls_w, cls_b):
    N = pos_features.shape[0]
    sub = N // 2 // _NB1
    grid2 = (2, sub)
    par2 = pltpu.CompilerParams(
        dimension_semantics=("parallel", "arbitrary"))
    grid = (N // _NB,)
    par = pltpu.CompilerParams(dimension_semantics=("parallel",))

    def _rows2(shape):
        return pl.BlockSpec(
            shape, lambda i, j: (i * sub + j,) + (0,) * (len(shape) - 1))

    def _whole2(shape):
        return pl.BlockSpec(shape, lambda i, j: (0,) * len(shape))

    def _tap2(shape):
        return pl.BlockSpec((1, 1) + shape[2:], lambda i, j: (1, 1, 0, 0))

    def _rows(shape):
        return pl.BlockSpec(shape, lambda i: (i,) + (0,) * (len(shape) - 1))

    def _whole(shape):
        return pl.BlockSpec(shape, lambda i: (0,) * len(shape))

    x_im = im_patches.reshape(N, 64)
    cws = [_conv_big_weight(pix_0_w, 8),
           _conv_big_weight(pix_1_w, 4),
           _conv_big_weight(pix_2_w, 2)]
    cbs = [pix_0_b.reshape(1, 32), pix_1_b.reshape(1, 64),
           pix_2_b.reshape(1, 128)]
    lstm_w = [bf_0_w, bf_1_w, bf_2_w, bf_3_w,
              br_0_w, br_1_w, br_2_w, br_3_w]
    lstm_b = [b.reshape(1, 256) for b in (bf_0_b, bf_1_b, bf_2_b, bf_3_b,
                                          br_0_b, br_1_b, br_2_b, br_3_b)]
    mlp = [pos_mlp_0_w, pos_mlp_0_b.reshape(1, -1),
           pos_mlp_1_w, pos_mlp_1_b.reshape(1, -1),
           pos_mlp_2_w, pos_mlp_2_b.reshape(1, -1),
           pos_mlp_3_w, pos_mlp_3_b.reshape(1, -1)]

    args1 = [x_im, pos_features,
             cws[0], cbs[0], cws[1], cbs[1], cws[2], cbs[2]]
    for w, b in zip(lstm_w, lstm_b):
        args1 += [w, b]
    args1 += mlp
    specs1 = [_rows2((_NB1, 64)), _rows2((_NB1, 128))]
    specs1 += [_tap2(a.shape) if a.ndim == 4 else _whole2(a.shape)
               for a in args1[2:]]
    cond, p0 = pl.pallas_call(
        _encode_body,
        grid=grid2,
        in_specs=specs1,
        out_specs=[_rows2((_NB1, 128)), _rows2((_NB1, 32))],
        out_shape=[jax.ShapeDtypeStruct((N, 128), jnp.bfloat16),
                   jax.ShapeDtypeStruct((N, 32), jnp.float32)],
        compiler_params=par2,
    )(*args1)

    p1, uc = pl.pallas_call(
        functools.partial(_gcn1_body, nb=_NB),
        grid=grid,
        in_specs=[_rows((_NB, N)), _whole((N, 32)), _whole((N, 128)),
                  _whole((160, 256)), _whole((1, 256)), _whole((256, 32)),
                  _whole((1, 32))],
        out_specs=[_rows((_NB, 32)), _rows((_NB, 128))],
        out_shape=[jax.ShapeDtypeStruct((N, 32), jnp.float32),
                   jax.ShapeDtypeStruct((N, 128), jnp.bfloat16)],
        compiler_params=par,
    )(A, p0, cond, gcn1_mw, gcn1_mb.reshape(1, 256), gcn1w_mw,
      gcn1w_mb.reshape(1, 32))

    out = pl.pallas_call(
        functools.partial(_gcn2_body, nb=_NB),
        grid=grid,
        in_specs=[_rows((_NB, N)), _whole((N, 32)), _rows((_NB, 128)),
                  _whole((160, 256)), _whole((1, 256)), _whole((256, 32)),
                  _whole((1, 32)), _whole((32, 128)), _whole((1, 128))],
        out_specs=_rows((_NB, 128)),
        out_shape=jax.ShapeDtypeStruct((N, 128), jnp.float32),
        compiler_params=par,
    )(A, p1, uc, gcn2_mw, gcn2_mb.reshape(1, 256), gcn2w_mw,
      gcn2w_mb.reshape(1, 32), cls_w, cls_b.reshape(1, 128))
    return out
